# baseline (speedup 1.0000x reference)
import math
import functools

import jax
import jax.numpy as jnp
import numpy as np
from jax import lax
from jax.experimental import pallas as pl
from jax.experimental.pallas import tpu as pltpu
from jax.experimental.pallas import tpu_sc as plsc

VOCAB = 100000
D_MODEL = 128
SEQ = 200
BATCH = 4096

NUM_CORES = 2
NUM_SUBCORES = 16
NUM_WORKERS = NUM_CORES * NUM_SUBCORES

TOKENS = BATCH * SEQ
TOK_PER_W = TOKENS // NUM_WORKERS
SEQ_PER_W = TOK_PER_W // SEQ
NSLOTS = 4
SEQS_PER_CHUNK = 1
CH = SEQS_PER_CHUNK * SEQ
CHUNKS_PER_W = TOK_PER_W // CH
GATHER_SPLITS = [
    (s * SEQ + off, n) for s in range(SEQS_PER_CHUNK) for off, n in ((0, 128), (128, 72))
]
NP = len(GATHER_SPLITS)


def _positional_encoding():
    position = np.arange(0, SEQ, dtype=np.float64)[:, None]
    div_term = np.exp(
        np.arange(0, D_MODEL, 2, dtype=np.float64) * -(math.log(10000.0) / D_MODEL)
    )
    pe = np.zeros((SEQ, D_MODEL), dtype=np.float32)
    pe[:, 0::2] = np.sin(position * div_term).astype(np.float32)
    pe[:, 1::2] = np.cos(position * div_term).astype(np.float32)
    return pe


@functools.cache
def _build_emb_kernel():
    mesh = plsc.VectorSubcoreMesh(
        core_axis_name="c",
        subcore_axis_name="s",
        num_cores=NUM_CORES,
        num_subcores=NUM_SUBCORES,
    )
    return functools.partial(
        pl.kernel,
        out_type=jax.ShapeDtypeStruct((TOKENS, D_MODEL), jnp.float32),
        mesh=mesh,
        scratch_types=[
            pltpu.VMEM_SHARED((SEQ, D_MODEL), jnp.float32),
            *([pltpu.VMEM((CH,), jnp.int32)] * NSLOTS),
            *([pltpu.VMEM((CH, D_MODEL), jnp.float32)] * NSLOTS),
            *([pltpu.SemaphoreType.DMA] * (NSLOTS * NP)),
            *([pltpu.SemaphoreType.DMA] * NSLOTS),
            *([pltpu.SemaphoreType.DMA] * (NSLOTS * NP)),
            *([pltpu.SemaphoreType.DMA] * (NSLOTS * NP)),
        ],
    )(_emb_body)


def _emb_body(x_hbm, emb_hbm, pe_hbm, out_hbm, pe_sh, *refs):
    cid = lax.axis_index("c")
    sid = lax.axis_index("s")
    wid = sid * NUM_CORES + cid
    base0 = wid * TOK_PER_W

    idx = refs[0:NSLOTS]
    rows = refs[NSLOTS:2 * NSLOTS]
    sems = refs[2 * NSLOTS:]

    def chop(seq, n):
        return [seq[j * n:(j + 1) * n] for j in range(NSLOTS)]

    sem_pre = chop(sems[0:NSLOTS * NP], NP)
    sem_idx = sems[NSLOTS * NP:NSLOTS * NP + NSLOTS]
    sem_g = chop(sems[NSLOTS * NP + NSLOTS:NSLOTS * (2 * NP + 1)], NP)
    sem_out = chop(sems[NSLOTS * (2 * NP + 1):], NP)

    @pl.when(sid == 0)
    def _():
        pltpu.sync_copy(pe_hbm, pe_sh)

    plsc.subcore_barrier()

    @pl.loop(0, CHUNKS_PER_W // NSLOTS)
    def _it(i):
        pres = []
        idxs = []
        for b in range(NSLOTS):
            base = base0 + (NSLOTS * i + b) * CH
            for k, (off, n) in enumerate(GATHER_SPLITS):
                @pl.when(i >= 1)
                def _(b=b, k=k, off=off, n=n, base=base):
                    pltpu.make_async_copy(
                        rows[b].at[pl.ds(off, n)],
                        out_hbm.at[pl.ds(base + off, n)],
                        sem_out[b][k],
                    ).wait()

            pres.append([
                pltpu.async_copy(
                    pe_sh.at[pl.ds(off % SEQ, n)],
                    rows[b].at[pl.ds(off, n)],
                    sem_pre[b][k],
                )
                for k, (off, n) in enumerate(GATHER_SPLITS)
            ])
            idxs.append(
                pltpu.async_copy(x_hbm.at[pl.ds(base, CH)], idx[b], sem_idx[b])
            )

        gathers = []
        for b in range(NSLOTS):
            idxs[b].wait()
            gs = []
            for k, (off, n) in enumerate(GATHER_SPLITS):
                pres[b][k].wait()
                gs.append(
                    pltpu.async_copy(
                        emb_hbm.at[idx[b].at[pl.ds(off, n)]],
                        rows[b].at[pl.ds(off, n)],
                        sem_g[b][k],
                        add=True,
                    )
                )
            gathers.append(gs)

        for b in range(NSLOTS):
            base = base0 + (NSLOTS * i + b) * CH
            for k, (off, n) in enumerate(GATHER_SPLITS):
                gathers[b][k].wait()
                pltpu.async_copy(
                    rows[b].at[pl.ds(off, n)],
                    out_hbm.at[pl.ds(base + off, n)],
                    sem_out[b][k],
                )

    for b in range(NSLOTS):
        base = base0 + (CHUNKS_PER_W - NSLOTS + b) * CH
        for k, (off, n) in enumerate(GATHER_SPLITS):
            pltpu.make_async_copy(
                rows[b].at[pl.ds(off, n)],
                out_hbm.at[pl.ds(base + off, n)],
                sem_out[b][k],
            ).wait()


_PE = _positional_encoding()


def kernel(x, token_emb):
    out = _build_emb_kernel()(x.reshape(-1), token_emb, jnp.asarray(_PE))
    return out.reshape(BATCH, SEQ, D_MODEL)


# speedup vs baseline: 1.0201x; 1.0201x over previous
import math
import functools

import jax
import jax.numpy as jnp
import numpy as np
from jax import lax
from jax.experimental import pallas as pl
from jax.experimental.pallas import tpu as pltpu
from jax.experimental.pallas import tpu_sc as plsc

VOCAB = 100000
D_MODEL = 128
SEQ = 200
BATCH = 4096

NUM_CORES = 2
NUM_SUBCORES = 16
NUM_WORKERS = NUM_CORES * NUM_SUBCORES

TOKENS = BATCH * SEQ
TOK_PER_W = TOKENS // NUM_WORKERS
SEQ_PER_W = TOK_PER_W // SEQ
NSLOTS = 4
SEQS_PER_CHUNK = 1
CH = SEQS_PER_CHUNK * SEQ
CHUNKS_PER_W = TOK_PER_W // CH
GATHER_SPLITS = [
    (s * SEQ + off, n) for s in range(SEQS_PER_CHUNK) for off, n in ((0, 104), (104, 96))
]
NP = len(GATHER_SPLITS)


def _positional_encoding():
    position = np.arange(0, SEQ, dtype=np.float64)[:, None]
    div_term = np.exp(
        np.arange(0, D_MODEL, 2, dtype=np.float64) * -(math.log(10000.0) / D_MODEL)
    )
    pe = np.zeros((SEQ, D_MODEL), dtype=np.float32)
    pe[:, 0::2] = np.sin(position * div_term).astype(np.float32)
    pe[:, 1::2] = np.cos(position * div_term).astype(np.float32)
    return pe


@functools.cache
def _build_emb_kernel():
    mesh = plsc.VectorSubcoreMesh(
        core_axis_name="c",
        subcore_axis_name="s",
        num_cores=NUM_CORES,
        num_subcores=NUM_SUBCORES,
    )
    return functools.partial(
        pl.kernel,
        out_type=jax.ShapeDtypeStruct((TOKENS, D_MODEL), jnp.float32),
        mesh=mesh,
        scratch_types=[
            pltpu.VMEM_SHARED((SEQ, D_MODEL), jnp.float32),
            *([pltpu.VMEM((CH,), jnp.int32)] * NSLOTS),
            *([pltpu.VMEM((CH, D_MODEL), jnp.float32)] * NSLOTS),
            *([pltpu.SemaphoreType.DMA] * (NSLOTS * NP)),
            *([pltpu.SemaphoreType.DMA] * NSLOTS),
            *([pltpu.SemaphoreType.DMA] * (NSLOTS * NP)),
            *([pltpu.SemaphoreType.DMA] * (NSLOTS * NP)),
        ],
    )(_emb_body)


def _emb_body(x_hbm, emb_hbm, pe_hbm, out_hbm, pe_sh, *refs):
    cid = lax.axis_index("c")
    sid = lax.axis_index("s")
    wid = sid * NUM_CORES + cid
    base0 = wid * TOK_PER_W

    idx = refs[0:NSLOTS]
    rows = refs[NSLOTS:2 * NSLOTS]
    sems = refs[2 * NSLOTS:]

    def chop(seq, n):
        return [seq[j * n:(j + 1) * n] for j in range(NSLOTS)]

    sem_pre = chop(sems[0:NSLOTS * NP], NP)
    sem_idx = sems[NSLOTS * NP:NSLOTS * NP + NSLOTS]
    sem_g = chop(sems[NSLOTS * NP + NSLOTS:NSLOTS * (2 * NP + 1)], NP)
    sem_out = chop(sems[NSLOTS * (2 * NP + 1):], NP)

    @pl.when(sid == 0)
    def _():
        pltpu.sync_copy(pe_hbm, pe_sh)

    plsc.subcore_barrier()

    @pl.loop(0, CHUNKS_PER_W // NSLOTS)
    def _it(i):
        pres = []
        idxs = []
        for b in range(NSLOTS):
            base = base0 + (NSLOTS * i + b) * CH
            for k, (off, n) in enumerate(GATHER_SPLITS):
                @pl.when(i >= 1)
                def _(b=b, k=k, off=off, n=n, base=base):
                    pltpu.make_async_copy(
                        rows[b].at[pl.ds(off, n)],
                        out_hbm.at[pl.ds(base + off, n)],
                        sem_out[b][k],
                    ).wait()

            pres.append([
                pltpu.async_copy(
                    pe_sh.at[pl.ds(off % SEQ, n)],
                    rows[b].at[pl.ds(off, n)],
                    sem_pre[b][k],
                )
                for k, (off, n) in enumerate(GATHER_SPLITS)
            ])
            idxs.append(
                pltpu.async_copy(x_hbm.at[pl.ds(base, CH)], idx[b], sem_idx[b])
            )

        gathers = []
        for b in range(NSLOTS):
            idxs[b].wait()
            gs = []
            for k, (off, n) in enumerate(GATHER_SPLITS):
                pres[b][k].wait()
                gs.append(
                    pltpu.async_copy(
                        emb_hbm.at[idx[b].at[pl.ds(off, n)]],
                        rows[b].at[pl.ds(off, n)],
                        sem_g[b][k],
                        add=True,
                    )
                )
            gathers.append(gs)

        for b in range(NSLOTS):
            base = base0 + (NSLOTS * i + b) * CH
            for k, (off, n) in enumerate(GATHER_SPLITS):
                gathers[b][k].wait()
                pltpu.async_copy(
                    rows[b].at[pl.ds(off, n)],
                    out_hbm.at[pl.ds(base + off, n)],
                    sem_out[b][k],
                )

    for b in range(NSLOTS):
        base = base0 + (CHUNKS_PER_W - NSLOTS + b) * CH
        for k, (off, n) in enumerate(GATHER_SPLITS):
            pltpu.make_async_copy(
                rows[b].at[pl.ds(off, n)],
                out_hbm.at[pl.ds(base + off, n)],
                sem_out[b][k],
            ).wait()


_PE = _positional_encoding()


def kernel(x, token_emb):
    out = _build_emb_kernel()(x.reshape(-1), token_emb, jnp.asarray(_PE))
    return out.reshape(BATCH, SEQ, D_MODEL)
